# baseline (speedup 1.0000x reference)
import jax
import jax.numpy as jnp
from jax.experimental import pallas as pl

_N = 1024 * 1024
_STEPS = 1
_BLOCK = _N // _STEPS


def _gen_kernel(w_ref, n_ref, o_ref):
    o_ref[...] = jax.nn.sigmoid((w_ref[...] - n_ref[...]) * 10.0)


def kernel(weights, noises):
    return pl.pallas_call(
        _gen_kernel,
        out_shape=jax.ShapeDtypeStruct((_N,), jnp.float32),
        grid=(_STEPS,),
        in_specs=[
            pl.BlockSpec((_BLOCK,), lambda i: (i,)),
            pl.BlockSpec((_BLOCK,), lambda i: (i,)),
        ],
        out_specs=pl.BlockSpec((_BLOCK,), lambda i: (i,)),
    )(weights, noises)


# speedup vs baseline: 1.0368x; 1.0368x over previous
import jax
import jax.numpy as jnp
from jax.experimental import pallas as pl
from jax.experimental.pallas import tpu as pltpu

_N = 1024 * 1024
_NCHUNK = 4
_CK = _N // _NCHUNK


def _body(w_hbm, n_hbm, o_hbm, w0, w1, n0, n1, o0, o1,
          sw0, sw1, sn0, sn1, so0, so1):
    wv = (w0, w1)
    nv = (n0, n1)
    ov = (o0, o1)
    sw = (sw0, sw1)
    sn = (sn0, sn1)
    so = (so0, so1)

    def issue_in(g):
        b = g & 1
        hw = pltpu.make_async_copy(w_hbm.at[pl.ds(g * _CK, _CK)], wv[b], sw[b])
        hn = pltpu.make_async_copy(n_hbm.at[pl.ds(g * _CK, _CK)], nv[b], sn[b])
        hw.start()
        hn.start()
        return hw, hn

    h_in = [None, None]
    h_out = [None, None]
    h_in[0] = issue_in(0)
    for g in range(_NCHUNK):
        b = g & 1
        if g + 1 < _NCHUNK:
            h_in[1 - b] = issue_in(g + 1)
        hw, hn = h_in[b]
        hw.wait()
        hn.wait()
        if h_out[b] is not None:
            h_out[b].wait()
        ov[b][...] = jax.nn.sigmoid((wv[b][...] - nv[b][...]) * 10.0)
        ho = pltpu.make_async_copy(ov[b], o_hbm.at[pl.ds(g * _CK, _CK)], so[b])
        ho.start()
        h_out[b] = ho
    h_out[(_NCHUNK - 2) & 1].wait()
    h_out[(_NCHUNK - 1) & 1].wait()


def kernel(weights, noises):
    return pl.pallas_call(
        _body,
        out_shape=jax.ShapeDtypeStruct((_N,), jnp.float32),
        in_specs=[
            pl.BlockSpec(memory_space=pl.ANY),
            pl.BlockSpec(memory_space=pl.ANY),
        ],
        out_specs=pl.BlockSpec(memory_space=pl.ANY),
        scratch_shapes=(
            [pltpu.VMEM((_CK,), jnp.float32) for _ in range(6)]
            + [pltpu.SemaphoreType.DMA for _ in range(6)]
        ),
    )(weights, noises)


# speedup vs baseline: 1.1987x; 1.1562x over previous
import jax
import jax.numpy as jnp
from jax.experimental import pallas as pl
from jax.experimental.pallas import tpu as pltpu

_N = 1024 * 1024
_NCHUNK = 2
_CK = _N // _NCHUNK


def _body(w_hbm, n_hbm, o_hbm, w0, w1, n0, n1, o0, o1,
          sw0, sw1, sn0, sn1, so0, so1):
    wv = (w0, w1)
    nv = (n0, n1)
    ov = (o0, o1)
    sw = (sw0, sw1)
    sn = (sn0, sn1)
    so = (so0, so1)

    def issue_in(g):
        b = g & 1
        hw = pltpu.make_async_copy(w_hbm.at[pl.ds(g * _CK, _CK)], wv[b], sw[b])
        hn = pltpu.make_async_copy(n_hbm.at[pl.ds(g * _CK, _CK)], nv[b], sn[b])
        hw.start()
        hn.start()
        return hw, hn

    h_in = [None, None]
    h_out = [None, None]
    h_in[0] = issue_in(0)
    for g in range(_NCHUNK):
        b = g & 1
        if g + 1 < _NCHUNK:
            h_in[1 - b] = issue_in(g + 1)
        hw, hn = h_in[b]
        hw.wait()
        hn.wait()
        if h_out[b] is not None:
            h_out[b].wait()
        ov[b][...] = jax.nn.sigmoid((wv[b][...] - nv[b][...]) * 10.0)
        ho = pltpu.make_async_copy(ov[b], o_hbm.at[pl.ds(g * _CK, _CK)], so[b])
        ho.start()
        h_out[b] = ho
    h_out[(_NCHUNK - 2) & 1].wait()
    h_out[(_NCHUNK - 1) & 1].wait()


def kernel(weights, noises):
    return pl.pallas_call(
        _body,
        out_shape=jax.ShapeDtypeStruct((_N,), jnp.float32),
        in_specs=[
            pl.BlockSpec(memory_space=pl.ANY),
            pl.BlockSpec(memory_space=pl.ANY),
        ],
        out_specs=pl.BlockSpec(memory_space=pl.ANY),
        scratch_shapes=(
            [pltpu.VMEM((_CK,), jnp.float32) for _ in range(6)]
            + [pltpu.SemaphoreType.DMA for _ in range(6)]
        ),
    )(weights, noises)


# speedup vs baseline: 1.2916x; 1.0775x over previous
import jax
import jax.numpy as jnp
from jax.experimental import pallas as pl
from jax.experimental.pallas import tpu as pltpu

_N = 1024 * 1024
_U = _N // 16
_CHUNKS = [5 * _U, 4 * _U, 3 * _U, 2 * _U, _U, _U]
_NCH = len(_CHUNKS)
_OFFS = [sum(_CHUNKS[:g]) for g in range(_NCH)]


def _body(w_hbm, n_hbm, o_hbm, *scr):
    wv = scr[0:_NCH]
    nv = scr[_NCH:2 * _NCH]
    ov = scr[2 * _NCH:3 * _NCH]
    sw = scr[3 * _NCH:4 * _NCH]
    sn = scr[4 * _NCH:5 * _NCH]
    so = scr[5 * _NCH:6 * _NCH]

    h_in = []
    for g in range(_NCH):
        hw = pltpu.make_async_copy(
            w_hbm.at[pl.ds(_OFFS[g], _CHUNKS[g])], wv[g], sw[g])
        hn = pltpu.make_async_copy(
            n_hbm.at[pl.ds(_OFFS[g], _CHUNKS[g])], nv[g], sn[g])
        hw.start()
        hn.start()
        h_in.append((hw, hn))

    h_out = []
    for g in range(_NCH):
        hw, hn = h_in[g]
        hw.wait()
        hn.wait()
        ov[g][...] = jax.nn.sigmoid((wv[g][...] - nv[g][...]) * 10.0)
        ho = pltpu.make_async_copy(
            ov[g], o_hbm.at[pl.ds(_OFFS[g], _CHUNKS[g])], so[g])
        ho.start()
        h_out.append(ho)
    for ho in h_out:
        ho.wait()


def kernel(weights, noises):
    return pl.pallas_call(
        _body,
        out_shape=jax.ShapeDtypeStruct((_N,), jnp.float32),
        in_specs=[
            pl.BlockSpec(memory_space=pl.ANY),
            pl.BlockSpec(memory_space=pl.ANY),
        ],
        out_specs=pl.BlockSpec(memory_space=pl.ANY),
        scratch_shapes=(
            [pltpu.VMEM((c,), jnp.float32) for c in _CHUNKS] * 3
            + [pltpu.SemaphoreType.DMA for _ in range(3 * _NCH)]
        ),
    )(weights, noises)
